# baseline (speedup 1.0000x reference)
import functools

import jax
import jax.numpy as jnp
from jax import lax
from jax.experimental import pallas as pl
from jax.experimental.pallas import tpu as pltpu
from jax.experimental.pallas import tpu_sc as plsc

B, S, H = 4096, 200, 64
N = B * S
VOCAB = 100000
AGE_VOCAB = 120
TYPE_VOCAB = 2
EPS = 1e-12

NC, NS = 2, 16
NW = NC * NS
PER_W = N // NW
PAIR = 1024
HALF = 512
NPAIR = PER_W // PAIR
KSUB = 4


def _gather_body(concept_hbm, cidx_hbm, out_hbm, idx_v, buf0, buf1,
                 sg0, sg1, sw0, sw1):
    wid = lax.axis_index("s") * NC + lax.axis_index("c")

    def pair_body(p, _):
        base = pl.multiple_of(wid * PER_W + p * PAIR, 8)
        krow = pl.multiple_of(base // 128, 8)
        pltpu.sync_copy(cidx_hbm.at[pl.ds(krow, 8)], idx_v)

        g0 = [pltpu.async_copy(concept_hbm.at[idx_v.at[k]],
                               buf0.at[pl.ds(k * 128, 128)], sg0)
              for k in range(KSUB)]
        g1 = [pltpu.async_copy(concept_hbm.at[idx_v.at[KSUB + k]],
                               buf1.at[pl.ds(k * 128, 128)], sg1)
              for k in range(KSUB)]
        for cp in g0:
            cp.wait()
        w0 = pltpu.async_copy(buf0, out_hbm.at[pl.ds(base, HALF)], sw0)
        for cp in g1:
            cp.wait()
        w1 = pltpu.async_copy(buf1, out_hbm.at[pl.ds(base + HALF, HALF)], sw1)
        w0.wait()
        w1.wait()
        return 0

    lax.fori_loop(0, NPAIR, pair_body, 0)


_sc_gather = functools.partial(
    pl.kernel,
    out_type=jax.ShapeDtypeStruct((N, H), jnp.bfloat16),
    mesh=plsc.VectorSubcoreMesh(core_axis_name="c", subcore_axis_name="s"),
    compiler_params=pltpu.CompilerParams(
        needs_layout_passes=False, use_tc_tiling_on_sc=False),
    scratch_types=[
        pltpu.VMEM((8, 128), jnp.int32),
        pltpu.VMEM((HALF, H), jnp.bfloat16),
        pltpu.VMEM((HALF, H), jnp.bfloat16),
        pltpu.SemaphoreType.DMA,
        pltpu.SemaphoreType.DMA,
        pltpu.SemaphoreType.DMA,
        pltpu.SemaphoreType.DMA,
    ],
)(_gather_body)


TB = 8192
NTB = N // TB
IDR = TB // 128


def _ln_body(g_ref, pos_ref, tt_ref, age_ref, seg_ref, gam_ref, bet_ref,
             o_ref):
    pos3 = pos_ref[...][:, :, None]
    tt3 = tt_ref[...][:, :, None]

    iota3 = lax.broadcasted_iota(jnp.int32, (IDR, 128, AGE_VOCAB), 2)
    onehot = (pos3 == iota3).astype(jnp.bfloat16).reshape(TB, AGE_VOCAB)
    age_f = age_ref[...]
    age_hi = age_f.astype(jnp.bfloat16)
    age_lo = (age_f - age_hi.astype(jnp.float32)).astype(jnp.bfloat16)
    dn = (((1,), (0,)), ((), ()))
    age_part = (
        lax.dot_general(onehot, age_hi, dn,
                        preferred_element_type=jnp.float32)
        + lax.dot_general(onehot, age_lo, dn,
                          preferred_element_type=jnp.float32))

    s = seg_ref[...]
    seg_part = jnp.where(tt3 == 0, s[0:1, :][None], s[1:2, :][None])
    seg_part = seg_part.reshape(TB, H)

    x = g_ref[...].astype(jnp.float32) + age_part + seg_part
    mean = jnp.mean(x, axis=-1, keepdims=True)
    cx = x - mean
    var = jnp.mean(cx * cx, axis=-1, keepdims=True)
    y = cx * lax.rsqrt(var + EPS)
    o_ref[...] = y * gam_ref[...] + bet_ref[...]


_tc_ln = pl.pallas_call(
    _ln_body,
    grid=(NTB,),
    in_specs=[
        pl.BlockSpec((TB, H), lambda i: (i, 0)),
        pl.BlockSpec((IDR, 128), lambda i: (i, 0)),
        pl.BlockSpec((IDR, 128), lambda i: (i, 0)),
        pl.BlockSpec((AGE_VOCAB, H), lambda i: (0, 0)),
        pl.BlockSpec((TYPE_VOCAB, H), lambda i: (0, 0)),
        pl.BlockSpec((1, H), lambda i: (0, 0)),
        pl.BlockSpec((1, H), lambda i: (0, 0)),
    ],
    out_specs=pl.BlockSpec((TB, H), lambda i: (i, 0)),
    out_shape=jax.ShapeDtypeStruct((N, H), jnp.float32),
    compiler_params=pltpu.CompilerParams(
        dimension_semantics=("arbitrary",)),
)


@jax.jit
def kernel(input_ids, token_type_ids, position_ids, concept_table,
           age_table, segment_table, ln_gamma, ln_beta):
    cidx = input_ids.astype(jnp.int32).reshape(N // 128, 128)
    pos = position_ids.astype(jnp.int32).reshape(N // 128, 128)
    tt = token_type_ids.astype(jnp.int32).reshape(N // 128, 128)
    gathered = _sc_gather(concept_table.astype(jnp.bfloat16), cidx)
    out = _tc_ln(gathered, pos, tt, age_table, segment_table,
                 ln_gamma.reshape(1, H), ln_beta.reshape(1, H))
    return out.reshape(B, S, H)


# speedup vs baseline: 1.2380x; 1.2380x over previous
import functools

import jax
import jax.numpy as jnp
from jax import lax
from jax.experimental import pallas as pl
from jax.experimental.pallas import tpu as pltpu
from jax.experimental.pallas import tpu_sc as plsc

B, S, H = 4096, 200, 64
N = B * S
VOCAB = 100000
AGE_VOCAB = 120
TYPE_VOCAB = 2
EPS = 1e-12

NC, NS = 2, 16
NW = NC * NS
PER_W = N // NW
PAIR = 1024
HALF = 512
NPAIR = PER_W // PAIR
KSUB = 4


def _gather_body(concept_hbm, cidx_hbm, out_hbm, idx_v, buf0, buf1,
                 sg0, sg1, sw0, sw1):
    wid = lax.axis_index("s") * NC + lax.axis_index("c")

    def pair_body(p, _):
        base = pl.multiple_of(wid * PER_W + p * PAIR, 8)
        krow = pl.multiple_of(base // 128, 8)
        pltpu.sync_copy(cidx_hbm.at[pl.ds(krow, 8)], idx_v)

        g0 = [pltpu.async_copy(concept_hbm.at[idx_v.at[k]],
                               buf0.at[pl.ds(k * 128, 128)], sg0)
              for k in range(KSUB)]
        g1 = [pltpu.async_copy(concept_hbm.at[idx_v.at[KSUB + k]],
                               buf1.at[pl.ds(k * 128, 128)], sg1)
              for k in range(KSUB)]
        for cp in g0:
            cp.wait()
        w0 = pltpu.async_copy(buf0, out_hbm.at[pl.ds(base, HALF)], sw0)
        for cp in g1:
            cp.wait()
        w1 = pltpu.async_copy(buf1, out_hbm.at[pl.ds(base + HALF, HALF)], sw1)
        w0.wait()
        w1.wait()
        return 0

    lax.fori_loop(0, NPAIR, pair_body, 0)


_sc_gather = functools.partial(
    pl.kernel,
    out_type=jax.ShapeDtypeStruct((N, H), jnp.float32),
    mesh=plsc.VectorSubcoreMesh(core_axis_name="c", subcore_axis_name="s"),
    compiler_params=pltpu.CompilerParams(
        needs_layout_passes=False, use_tc_tiling_on_sc=False),
    scratch_types=[
        pltpu.VMEM((8, 128), jnp.int32),
        pltpu.VMEM((HALF, H), jnp.float32),
        pltpu.VMEM((HALF, H), jnp.float32),
        pltpu.SemaphoreType.DMA,
        pltpu.SemaphoreType.DMA,
        pltpu.SemaphoreType.DMA,
        pltpu.SemaphoreType.DMA,
    ],
)(_gather_body)


TB = 8192
NTB = N // TB
IDR = TB // 128


CVOC = AGE_VOCAB * TYPE_VOCAB


def _ln_body(g_ref, pos_ref, tt_ref, age_ref, seg_ref, gam_ref, bet_ref,
             o_ref):
    cid3 = (pos_ref[...] * 2 + tt_ref[...])[:, :, None]
    iota3 = lax.broadcasted_iota(jnp.int32, (IDR, 128, CVOC), 2)
    onehot = (cid3 == iota3).astype(jnp.bfloat16).reshape(TB, CVOC)

    comb = (age_ref[...][:, None, :]
            + seg_ref[...][None, :, :]).reshape(CVOC, H)
    comb_hi = comb.astype(jnp.bfloat16)
    comb_lo = (comb - comb_hi.astype(jnp.float32)).astype(jnp.bfloat16)
    dn = (((1,), (0,)), ((), ()))
    emb_part = (
        lax.dot_general(onehot, comb_hi, dn,
                        preferred_element_type=jnp.float32)
        + lax.dot_general(onehot, comb_lo, dn,
                          preferred_element_type=jnp.float32))

    x = g_ref[...] + emb_part
    mean = jnp.mean(x, axis=-1, keepdims=True)
    cx = x - mean
    var = jnp.mean(cx * cx, axis=-1, keepdims=True)
    y = cx * lax.rsqrt(var + EPS)
    o_ref[...] = y * gam_ref[...] + bet_ref[...]


_tc_ln = pl.pallas_call(
    _ln_body,
    grid=(NTB,),
    in_specs=[
        pl.BlockSpec((TB, H), lambda i: (i, 0)),
        pl.BlockSpec((IDR, 128), lambda i: (i, 0)),
        pl.BlockSpec((IDR, 128), lambda i: (i, 0)),
        pl.BlockSpec((AGE_VOCAB, H), lambda i: (0, 0)),
        pl.BlockSpec((TYPE_VOCAB, H), lambda i: (0, 0)),
        pl.BlockSpec((1, H), lambda i: (0, 0)),
        pl.BlockSpec((1, H), lambda i: (0, 0)),
    ],
    out_specs=pl.BlockSpec((TB, H), lambda i: (i, 0)),
    out_shape=jax.ShapeDtypeStruct((N, H), jnp.float32),
    compiler_params=pltpu.CompilerParams(
        dimension_semantics=("arbitrary",)),
)


@jax.jit
def kernel(input_ids, token_type_ids, position_ids, concept_table,
           age_table, segment_table, ln_gamma, ln_beta):
    cidx = input_ids.astype(jnp.int32).reshape(N // 128, 128)
    pos = position_ids.astype(jnp.int32).reshape(N // 128, 128)
    tt = token_type_ids.astype(jnp.int32).reshape(N // 128, 128)
    gathered = _sc_gather(concept_table, cidx)
    out = _tc_ln(gathered, pos, tt, age_table, segment_table,
                 ln_gamma.reshape(1, H), ln_beta.reshape(1, H))
    return out.reshape(B, S, H)


# speedup vs baseline: 1.2491x; 1.0089x over previous
import functools

import jax
import jax.numpy as jnp
from jax import lax
from jax.experimental import pallas as pl
from jax.experimental.pallas import tpu as pltpu
from jax.experimental.pallas import tpu_sc as plsc

B, S, H = 4096, 200, 64
N = B * S
VOCAB = 100000
AGE_VOCAB = 120
TYPE_VOCAB = 2
EPS = 1e-12

NC, NS = 2, 16
NW = NC * NS
PER_W = N // NW
PAIR = 1024
HALF = 512
NPAIR = PER_W // PAIR
KSUB = 4


def _gather_body(concept_hbm, cidx_hbm, out_hbm, idx_v, buf0, buf1,
                 sg0, sg1, sw0, sw1):
    wid = lax.axis_index("s") * NC + lax.axis_index("c")

    def pair_body(p, _):
        base = pl.multiple_of(wid * PER_W + p * PAIR, 8)
        krow = pl.multiple_of(base // 128, 8)
        pltpu.sync_copy(cidx_hbm.at[pl.ds(krow, 8)], idx_v)

        g0 = [pltpu.async_copy(concept_hbm.at[idx_v.at[k]],
                               buf0.at[pl.ds(k * 128, 128)], sg0)
              for k in range(KSUB)]
        g1 = [pltpu.async_copy(concept_hbm.at[idx_v.at[KSUB + k]],
                               buf1.at[pl.ds(k * 128, 128)], sg1)
              for k in range(KSUB)]
        for cp in g0:
            cp.wait()
        w0 = pltpu.async_copy(buf0, out_hbm.at[pl.ds(base, HALF)], sw0)
        for cp in g1:
            cp.wait()
        w1 = pltpu.async_copy(buf1, out_hbm.at[pl.ds(base + HALF, HALF)], sw1)
        w0.wait()
        w1.wait()
        return 0

    lax.fori_loop(0, NPAIR, pair_body, 0)


_sc_gather = functools.partial(
    pl.kernel,
    out_type=jax.ShapeDtypeStruct((N, H), jnp.float32),
    mesh=plsc.VectorSubcoreMesh(core_axis_name="c", subcore_axis_name="s"),
    compiler_params=pltpu.CompilerParams(
        needs_layout_passes=False, use_tc_tiling_on_sc=False),
    scratch_types=[
        pltpu.VMEM((8, 128), jnp.int32),
        pltpu.VMEM((HALF, H), jnp.float32),
        pltpu.VMEM((HALF, H), jnp.float32),
        pltpu.SemaphoreType.DMA,
        pltpu.SemaphoreType.DMA,
        pltpu.SemaphoreType.DMA,
        pltpu.SemaphoreType.DMA,
    ],
)(_gather_body)


TB = 16384
NTB = N // TB
IDR = TB // 128


CVOC = AGE_VOCAB * TYPE_VOCAB


def _ln_body(g_ref, pos_ref, tt_ref, age_ref, seg_ref, gam_ref, bet_ref,
             o_ref):
    cid3 = (pos_ref[...] * 2 + tt_ref[...])[:, :, None]
    iota3 = lax.broadcasted_iota(jnp.int32, (IDR, 128, CVOC), 2)
    onehot = (cid3 == iota3).astype(jnp.bfloat16).reshape(TB, CVOC)

    comb = (age_ref[...][:, None, :]
            + seg_ref[...][None, :, :]).reshape(CVOC, H)
    comb_hi = comb.astype(jnp.bfloat16)
    comb_lo = (comb - comb_hi.astype(jnp.float32)).astype(jnp.bfloat16)
    dn = (((1,), (0,)), ((), ()))
    emb_part = (
        lax.dot_general(onehot, comb_hi, dn,
                        preferred_element_type=jnp.float32)
        + lax.dot_general(onehot, comb_lo, dn,
                          preferred_element_type=jnp.float32))

    x = g_ref[...] + emb_part
    mean = jnp.mean(x, axis=-1, keepdims=True)
    cx = x - mean
    var = jnp.mean(cx * cx, axis=-1, keepdims=True)
    y = cx * lax.rsqrt(var + EPS)
    o_ref[...] = y * gam_ref[...] + bet_ref[...]


_tc_ln = pl.pallas_call(
    _ln_body,
    grid=(NTB,),
    in_specs=[
        pl.BlockSpec((TB, H), lambda i: (i, 0)),
        pl.BlockSpec((IDR, 128), lambda i: (i, 0)),
        pl.BlockSpec((IDR, 128), lambda i: (i, 0)),
        pl.BlockSpec((AGE_VOCAB, H), lambda i: (0, 0)),
        pl.BlockSpec((TYPE_VOCAB, H), lambda i: (0, 0)),
        pl.BlockSpec((1, H), lambda i: (0, 0)),
        pl.BlockSpec((1, H), lambda i: (0, 0)),
    ],
    out_specs=pl.BlockSpec((TB, H), lambda i: (i, 0)),
    out_shape=jax.ShapeDtypeStruct((N, H), jnp.float32),
    compiler_params=pltpu.CompilerParams(
        dimension_semantics=("arbitrary",)),
)


@jax.jit
def kernel(input_ids, token_type_ids, position_ids, concept_table,
           age_table, segment_table, ln_gamma, ln_beta):
    cidx = input_ids.astype(jnp.int32).reshape(N // 128, 128)
    pos = position_ids.astype(jnp.int32).reshape(N // 128, 128)
    tt = token_type_ids.astype(jnp.int32).reshape(N // 128, 128)
    gathered = _sc_gather(concept_table, cidx)
    out = _tc_ln(gathered, pos, tt, age_table, segment_table,
                 ln_gamma.reshape(1, H), ln_beta.reshape(1, H))
    return out.reshape(B, S, H)
